# baseline (speedup 1.0000x reference)
import functools
import jax
import jax.numpy as jnp
from jax import lax
from jax.experimental import pallas as pl
from jax.experimental.pallas import tpu as pltpu
from jax.experimental.pallas import tpu_sc as plsc


def _full16(v):
    return jnp.full((16,), v, jnp.int32)


def _sc_body(x_hbm, w_hbm, b_hbm, tod_hbm, dow_hbm, adp_hbm, out_hbm,
             w_v, b_v, tod_v, dow_v, adp_v, x_v, row_v,
             *, B, L, N, n_per_w, E, A, steps_per_day):
    NC = 2
    wid = lax.axis_index("s") * NC + lax.axis_index("c")
    n0 = wid * n_per_w
    OUT = 3 * E + A

    pltpu.sync_copy(w_hbm, w_v)
    pltpu.sync_copy(b_hbm, b_v)
    pltpu.sync_copy(tod_hbm, tod_v)
    pltpu.sync_copy(dow_hbm, dow_v)
    for l in range(L):
        pltpu.sync_copy(
            adp_hbm.at[pl.ds(l * N * A + n0 * A, n_per_w * A)],
            adp_v.at[pl.ds(l * n_per_w * A, n_per_w * A)],
        )

    i16 = lax.iota(jnp.int32, 16)
    n_groups = n_per_w // 16

    def lbody(l, carry):
        def gadp(g, c0):
            rows = g * 16 + i16
            abase = l * n_per_w * A + rows * A

            def cadp(c, c1):
                v = plsc.load_gather(adp_v, [abase + c])
                plsc.store_scatter(row_v, [rows, jnp.full((16,), 3 * E, jnp.int32) + c], v)
                return c1

            lax.fori_loop(0, A, cadp, 0)
            return c0

        lax.fori_loop(0, n_groups, gadp, 0)

        def bbody(b, c2):
            base = ((b * L + l) * N + n0) * 3
            pltpu.sync_copy(x_hbm.at[pl.ds(base, n_per_w * 3)], x_v)
            def gbody(g, c3):
                rows = g * 16 + i16
                xofs = rows * 3
                x0 = plsc.load_gather(x_v, [xofs])
                x1 = plsc.load_gather(x_v, [xofs + 1])
                x2 = plsc.load_gather(x_v, [xofs + 2])
                ti = (x1 * jnp.float32(steps_per_day)).astype(jnp.int32) * E
                di = x2.astype(jnp.int32) * E

                def jbody(j, c4):
                    jv = jnp.full((16,), j, jnp.int32)
                    w0 = plsc.load_gather(w_v, [jv])
                    w1 = plsc.load_gather(w_v, [jv + E])
                    w2 = plsc.load_gather(w_v, [jv + 2 * E])
                    bj = plsc.load_gather(b_v, [jv])
                    xpj = x0 * w0 + x1 * w1 + x2 * w2 + bj
                    plsc.store_scatter(row_v, [rows, jv], xpj)
                    tj = plsc.load_gather(tod_v, [ti + j])
                    plsc.store_scatter(row_v, [rows, jv + E], tj)
                    dj = plsc.load_gather(dow_v, [di + j])
                    plsc.store_scatter(row_v, [rows, jv + 2 * E], dj)
                    return c4

                lax.fori_loop(0, E, jbody, 0)
                return c3

            lax.fori_loop(0, n_groups, gbody, 0)
            pltpu.sync_copy(row_v, out_hbm.at[b, l, pl.ds(n0, n_per_w), :])
            return c2

        lax.fori_loop(0, B, bbody, 0)
        return carry

    lax.fori_loop(0, L, lbody, 0)


def kernel(x, W_in, b_in, tod_table, dow_table, adp):
    B, L, N, D = x.shape
    E = W_in.shape[1]
    A = adp.shape[-1]
    OUT = 3 * E + A
    NW = 32
    n_per_w = N // NW

    mesh = plsc.VectorSubcoreMesh(core_axis_name="c", subcore_axis_name="s")
    f = functools.partial(
        pl.kernel,
        out_type=jax.ShapeDtypeStruct((B, L, N, OUT), jnp.float32),
        mesh=mesh,
        compiler_params=pltpu.CompilerParams(needs_layout_passes=False),
        scratch_types=[
            pltpu.VMEM((D * E,), jnp.float32),
            pltpu.VMEM((E,), jnp.float32),
            pltpu.VMEM((tod_table.shape[0] * E,), jnp.float32),
            pltpu.VMEM((dow_table.shape[0] * E,), jnp.float32),
            pltpu.VMEM((L * n_per_w * A,), jnp.float32),
            pltpu.VMEM((n_per_w * D,), jnp.float32),
            pltpu.VMEM((n_per_w, OUT), jnp.float32),
        ],
    )(functools.partial(_sc_body, B=B, L=L, N=N, n_per_w=n_per_w, E=E, A=A,
                        steps_per_day=288))
    return f(x.reshape(-1), W_in.reshape(-1), b_in, tod_table.reshape(-1),
             dow_table.reshape(-1), adp.reshape(-1))


# speedup vs baseline: 2.7586x; 2.7586x over previous
import functools
import jax
import jax.numpy as jnp
from jax import lax
from jax.experimental import pallas as pl
from jax.experimental.pallas import tpu as pltpu


def _embed_body(x_ref, w_ref, b_ref, tab_ref, adp_ref, out_ref, *, steps_per_day, n_tod):
    xv = x_ref[0, 0]
    x1 = xv[:, 1:2]
    x2 = xv[:, 2:3]
    w = w_ref[...]
    xp = (
        xv[:, 0:1] * w[0:1, :]
        + x1 * w[1:2, :]
        + x2 * w[2:3, :]
        + b_ref[...]
    )

    n_rows = tab_ref.shape[0]
    ti = (x1 * jnp.float32(steps_per_day)).astype(jnp.int32)
    di = x2.astype(jnp.int32) + n_tod
    lanes = lax.broadcasted_iota(jnp.int32, (1, n_rows), 1)
    oh = ((ti == lanes) | (di == lanes)).astype(jnp.float32)
    emb = jnp.dot(oh, tab_ref[...], preferred_element_type=jnp.float32)

    out_ref[0, 0] = jnp.concatenate([xp, emb, adp_ref[0]], axis=-1)


def kernel(x, W_in, b_in, tod_table, dow_table, adp):
    B, L, N, D = x.shape
    E = W_in.shape[1]
    A = adp.shape[-1]
    OUT = E * 3 + A
    CHUNK = 1024
    assert N % CHUNK == 0

    b2 = b_in.reshape(1, E)
    n_tod = tod_table.shape[0]
    n_dow = dow_table.shape[0]
    tab = jnp.zeros((n_tod + n_dow, 2 * E), jnp.float32)
    tab = tab.at[:n_tod, :E].set(tod_table).at[n_tod:, E:].set(dow_table)

    grid = (L, N // CHUNK, B)
    return pl.pallas_call(
        functools.partial(_embed_body, steps_per_day=288, n_tod=n_tod),
        grid=grid,
        in_specs=[
            pl.BlockSpec((1, 1, CHUNK, D), lambda l, c, b: (b, l, c, 0)),
            pl.BlockSpec((D, E), lambda l, c, b: (0, 0)),
            pl.BlockSpec((1, E), lambda l, c, b: (0, 0)),
            pl.BlockSpec(tab.shape, lambda l, c, b: (0, 0)),
            pl.BlockSpec((1, CHUNK, A), lambda l, c, b: (l, c, 0)),
        ],
        out_specs=pl.BlockSpec((1, 1, CHUNK, OUT), lambda l, c, b: (b, l, c, 0)),
        out_shape=jax.ShapeDtypeStruct((B, L, N, OUT), jnp.float32),
    )(x, W_in, b2, tab, adp)


# speedup vs baseline: 3.0815x; 1.1171x over previous
import functools
import jax
import jax.numpy as jnp
from jax import lax
from jax.experimental import pallas as pl
from jax.experimental.pallas import tpu as pltpu


def _embed_body(x_ref, w_ref, b_ref, tab_ref, adp_ref, out_ref, *,
                steps_per_day, n_tod, lb):
    w = w_ref[...]
    n_rows = tab_ref.shape[0]
    lanes = lax.broadcasted_iota(jnp.int32, (1, n_rows), 1)
    for li in range(lb):
        xv = x_ref[0, li]
        x1 = xv[:, 1:2]
        x2 = xv[:, 2:3]
        xp = (
            xv[:, 0:1] * w[0:1, :]
            + x1 * w[1:2, :]
            + x2 * w[2:3, :]
            + b_ref[...]
        )

        ti = (x1 * jnp.float32(steps_per_day)).astype(jnp.int32)
        di = x2.astype(jnp.int32) + n_tod
        oh = ((ti == lanes) | (di == lanes)).astype(jnp.float32)
        emb = jnp.dot(oh, tab_ref[...], preferred_element_type=jnp.float32)

        out_ref[0, li] = jnp.concatenate([xp, emb, adp_ref[li]], axis=-1)


def kernel(x, W_in, b_in, tod_table, dow_table, adp):
    B, L, N, D = x.shape
    E = W_in.shape[1]
    A = adp.shape[-1]
    OUT = E * 3 + A
    CHUNK = 2048
    LB = 2
    assert N % CHUNK == 0 and L % LB == 0

    b2 = b_in.reshape(1, E)
    n_tod = tod_table.shape[0]
    n_dow = dow_table.shape[0]
    tab = jnp.zeros((n_tod + n_dow, 2 * E), jnp.float32)
    tab = tab.at[:n_tod, :E].set(tod_table).at[n_tod:, E:].set(dow_table)

    grid = (L // LB, N // CHUNK, B)
    return pl.pallas_call(
        functools.partial(_embed_body, steps_per_day=288, n_tod=n_tod, lb=LB),
        grid=grid,
        in_specs=[
            pl.BlockSpec((1, LB, CHUNK, D), lambda l, c, b: (b, l, c, 0)),
            pl.BlockSpec((D, E), lambda l, c, b: (0, 0)),
            pl.BlockSpec((1, E), lambda l, c, b: (0, 0)),
            pl.BlockSpec(tab.shape, lambda l, c, b: (0, 0)),
            pl.BlockSpec((LB, CHUNK, A), lambda l, c, b: (l, c, 0)),
        ],
        out_specs=pl.BlockSpec((1, LB, CHUNK, OUT), lambda l, c, b: (b, l, c, 0)),
        out_shape=jax.ShapeDtypeStruct((B, L, N, OUT), jnp.float32),
    )(x, W_in, b2, tab, adp)
